# baseline (speedup 1.0000x reference)
import functools

import jax
import jax.numpy as jnp
from jax import lax
from jax.experimental import pallas as pl
from jax.experimental.pallas import tpu as pltpu
from jax.experimental.pallas import tpu_sc as plsc

N = 100000
E = 3200000
D = 16
NC = 2
NS = 16
NW = NC * NS
ROWS_TOTAL = E // 128
ROWS_PER_WORKER = 781
TAIL_ROW0 = ROWS_PER_WORKER * NW
TAIL_ROWS = ROWS_TOTAL - TAIL_ROW0
CHUNK_ROWS = 11
CHUNK_E = CHUNK_ROWS * 128
N_CHUNKS = ROWS_PER_WORKER // CHUNK_ROWS
N_PAD = 100096
TILE_NODES = N_PAD // NS

_mesh = functools.partial(
    plsc.VectorSubcoreMesh, core_axis_name="c", subcore_axis_name="s",
    num_cores=NC, num_subcores=NS)

_sc_params = functools.partial(
    pltpu.CompilerParams, use_tc_tiling_on_sc=False,
    needs_layout_passes=False)


def _worker_base():
  c = lax.axis_index("c")
  s = lax.axis_index("s")
  wid = c * NS + s
  return c, s, wid, wid * ROWS_PER_WORKER


def _dega_body(src_hbm, dst_hbm, od_hbm, id_hbm,
               srcb, dstb, onesb, zerob, od_sh, id_sh, hsem):
  c, s, wid, row0 = _worker_base()

  @pl.loop(0, TILE_NODES // 16)
  def _z(i):
    zerob[pl.ds(i * 16, 16)] = jnp.zeros((16,), jnp.float32)

  for g in range(8):
    onesb[pl.ds(g * 16, 16)] = jnp.ones((16,), jnp.float32)

  node0 = s * TILE_NODES
  pltpu.sync_copy(zerob, od_sh.at[pl.ds(node0, TILE_NODES)])
  pltpu.sync_copy(zerob, id_sh.at[pl.ds(node0, TILE_NODES)])
  plsc.subcore_barrier()

  def _do_rows(r0, nrows):
    pltpu.sync_copy(src_hbm.at[pl.ds(r0, nrows), :],
                    srcb.at[pl.ds(0, nrows), :])
    pltpu.sync_copy(dst_hbm.at[pl.ds(r0, nrows), :],
                    dstb.at[pl.ds(0, nrows), :])
    descs = []
    for j in range(nrows):
      descs.append(pltpu.async_copy(onesb, od_sh.at[srcb.at[j]], hsem,
                                    add=True))
      descs.append(pltpu.async_copy(onesb, id_sh.at[dstb.at[j]], hsem,
                                    add=True))
    for d in descs:
      d.wait()

  @pl.loop(0, N_CHUNKS)
  def _chunk(ch):
    _do_rows(row0 + ch * CHUNK_ROWS, CHUNK_ROWS)

  @pl.when(wid == 0)
  def _tail():
    _do_rows(TAIL_ROW0, TAIL_ROWS)

  plsc.subcore_barrier()
  pltpu.sync_copy(od_sh.at[pl.ds(node0, TILE_NODES)],
                  od_hbm.at[c, pl.ds(node0, TILE_NODES)])
  pltpu.sync_copy(id_sh.at[pl.ds(node0, TILE_NODES)],
                  id_hbm.at[c, pl.ds(node0, TILE_NODES)])


def _make_dega():
  return pl.kernel(
      _dega_body,
      out_type=[
          jax.ShapeDtypeStruct((NC, N_PAD), jnp.float32),
          jax.ShapeDtypeStruct((NC, N_PAD), jnp.float32),
      ],
      mesh=_mesh(),
      compiler_params=_sc_params(),
      scratch_types=[
          pltpu.VMEM((CHUNK_ROWS, 128), jnp.int32),
          pltpu.VMEM((CHUNK_ROWS, 128), jnp.int32),
          pltpu.VMEM((128,), jnp.float32),
          pltpu.VMEM((TILE_NODES,), jnp.float32),
          pltpu.VMEM_SHARED((N_PAD,), jnp.float32),
          pltpu.VMEM_SHARED((N_PAD,), jnp.float32),
          pltpu.SemaphoreType.DMA,
      ],
  )


def _scat_body(src_hbm, dst_hbm, ew_hbm, h1_hbm, agg_hbm,
               srcb, dstb, ewb, rows, agg_sh, gsem, ssem):
  c, s, wid, row0 = _worker_base()

  zstep = TILE_NODES // 16

  @pl.loop(0, zstep)
  def _z(i):
    rows[i, :] = jnp.zeros((16,), jnp.float32)

  node0 = s * TILE_NODES
  for k in range(16):
    pltpu.sync_copy(rows.at[pl.ds(0, zstep), :],
                    agg_sh.at[pl.ds(node0 + k * zstep, zstep), :])
  plsc.subcore_barrier()

  def _do_rows(r0, nrows):
    pltpu.sync_copy(src_hbm.at[pl.ds(r0, nrows), :],
                    srcb.at[pl.ds(0, nrows), :])
    pltpu.sync_copy(dst_hbm.at[pl.ds(r0, nrows), :],
                    dstb.at[pl.ds(0, nrows), :])
    pltpu.sync_copy(ew_hbm.at[pl.ds(r0, nrows), :],
                    ewb.at[pl.ds(0, nrows), :])
    gds = [pltpu.async_copy(h1_hbm.at[srcb.at[j]],
                            rows.at[pl.ds(j * 128, 128), :], gsem)
           for j in range(nrows)]
    sds = []
    for j in range(nrows):
      gds[j].wait()

      @pl.loop(0, 8)
      def _scale(q):
        base = j * 128 + q * 16
        ew16 = ewb[j, pl.ds(q * 16, 16)]
        for i in range(16):
          rows[base + i, :] = rows[base + i, :] * ew16[i]

      sds.append(pltpu.async_copy(rows.at[pl.ds(j * 128, 128), :],
                                  agg_sh.at[dstb.at[j]], ssem, add=True))
    for d in sds:
      d.wait()

  @pl.loop(0, N_CHUNKS)
  def _chunk(ch):
    _do_rows(row0 + ch * CHUNK_ROWS, CHUNK_ROWS)

  @pl.when(wid == 0)
  def _tail():
    _do_rows(TAIL_ROW0, TAIL_ROWS)

  plsc.subcore_barrier()
  for k in range(16):
    pltpu.sync_copy(agg_sh.at[pl.ds(node0 + k * zstep, zstep), :],
                    agg_hbm.at[c].at[pl.ds(node0 + k * zstep, zstep), :])


def _make_scat():
  return pl.kernel(
      _scat_body,
      out_type=[
          jax.ShapeDtypeStruct((NC, N_PAD, D), jnp.float32),
      ],
      mesh=_mesh(),
      compiler_params=_sc_params(),
      scratch_types=[
          pltpu.VMEM((CHUNK_ROWS, 128), jnp.int32),
          pltpu.VMEM((CHUNK_ROWS, 128), jnp.int32),
          pltpu.VMEM((CHUNK_ROWS, 128), jnp.float32),
          pltpu.VMEM((CHUNK_E, D), jnp.float32),
          pltpu.VMEM_SHARED((N_PAD, D), jnp.float32),
          pltpu.SemaphoreType.DMA,
          pltpu.SemaphoreType.DMA,
      ],
  )


def _esum_body(src_hbm, dst_hbm, ew_hbm, h2_hbm, cd_hbm, part_hbm,
               srcb, dstb, ewb, h2b, cb, accb, h2_sh, c_sh, gsem):
  c, s, wid, row0 = _worker_base()

  node0 = s * TILE_NODES
  pltpu.sync_copy(h2_hbm.at[pl.ds(node0, TILE_NODES)],
                  h2_sh.at[pl.ds(node0, TILE_NODES)])
  pltpu.sync_copy(cd_hbm.at[pl.ds(node0, TILE_NODES)],
                  c_sh.at[pl.ds(node0, TILE_NODES)])
  plsc.subcore_barrier()

  def _do_rows(r0, nrows, acc):
    pltpu.sync_copy(src_hbm.at[pl.ds(r0, nrows), :],
                    srcb.at[pl.ds(0, nrows), :])
    pltpu.sync_copy(dst_hbm.at[pl.ds(r0, nrows), :],
                    dstb.at[pl.ds(0, nrows), :])
    pltpu.sync_copy(ew_hbm.at[pl.ds(r0, nrows), :],
                    ewb.at[pl.ds(0, nrows), :])
    cds = []
    for j in range(nrows):
      cds.append((pltpu.async_copy(h2_sh.at[srcb.at[j]],
                                   h2b.at[pl.ds(j * 128, 128)], gsem),
                  pltpu.async_copy(c_sh.at[dstb.at[j]],
                                   cb.at[pl.ds(j * 128, 128)], gsem)))
    for j in range(nrows):
      cds[j][0].wait()
      cds[j][1].wait()
      for g in range(8):
        sl = pl.ds(j * 128 + g * 16, 16)
        acc = acc + ewb[j, pl.ds(g * 16, 16)] * h2b[sl] * cb[sl]
    return acc

  def _chunk(ch, acc):
    return _do_rows(row0 + ch * CHUNK_ROWS, CHUNK_ROWS, acc)

  acc = lax.fori_loop(0, N_CHUNKS, _chunk, jnp.zeros((16,), jnp.float32))
  acc2 = _do_rows(TAIL_ROW0, TAIL_ROWS, acc)
  accb[...] = jnp.where(wid == 0, acc2, acc)
  pltpu.sync_copy(accb, part_hbm.at[c, s, :])


def _make_esum():
  return pl.kernel(
      _esum_body,
      out_type=[
          jax.ShapeDtypeStruct((NC, NS, 16), jnp.float32),
      ],
      mesh=_mesh(),
      compiler_params=_sc_params(),
      scratch_types=[
          pltpu.VMEM((CHUNK_ROWS, 128), jnp.int32),
          pltpu.VMEM((CHUNK_ROWS, 128), jnp.int32),
          pltpu.VMEM((CHUNK_ROWS, 128), jnp.float32),
          pltpu.VMEM((CHUNK_E,), jnp.float32),
          pltpu.VMEM((CHUNK_E,), jnp.float32),
          pltpu.VMEM((16,), jnp.float32),
          pltpu.VMEM_SHARED((N_PAD,), jnp.float32),
          pltpu.VMEM_SHARED((N_PAD,), jnp.float32),
          pltpu.SemaphoreType.DMA,
      ],
  )


_BR_E = 5120
_ROWS_B = _BR_E // 128


def _prep_body(ei_ref, rt_ref, af_ref, wemb_ref, bemb_ref,
               src_ref, dst_ref, ew_ref, nf_ref):
  src_ref[...] = ei_ref[0, :].reshape(_ROWS_B, 128)
  dst_ref[...] = ei_ref[1, :].reshape(_ROWS_B, 128)
  x = rt_ref[0, :]
  y = rt_ref[1, :]
  z = rt_ref[2, :]
  ew = jnp.exp(-(x * x + y * y + z * z))
  ew_ref[...] = ew.reshape(_ROWS_B, 128)
  i = pl.program_id(0)

  @pl.when(i < N // _BN)
  def _nf():
    nf_ref[...] = jnp.dot(af_ref[...], wemb_ref[...],
                          preferred_element_type=jnp.float32) + bemb_ref[...]


def _clampn(i):
  return (jnp.minimum(i, N // _BN - 1), 0)


def _tc_prep(edge_index, rt, af, wembp, bembp):
  return pl.pallas_call(
      _prep_body,
      grid=(E // _BR_E,),
      in_specs=[
          pl.BlockSpec((2, _BR_E), lambda i: (0, i)),
          pl.BlockSpec((3, _BR_E), lambda i: (0, i)),
          pl.BlockSpec((_BN, 128), _clampn),
          pl.BlockSpec((128, D), lambda i: (0, 0)),
          pl.BlockSpec((1, D), lambda i: (0, 0)),
      ],
      out_specs=[
          pl.BlockSpec((_ROWS_B, 128), lambda i: (i, 0)),
          pl.BlockSpec((_ROWS_B, 128), lambda i: (i, 0)),
          pl.BlockSpec((_ROWS_B, 128), lambda i: (i, 0)),
          pl.BlockSpec((_BN, D), _clampn),
      ],
      out_shape=[
          jax.ShapeDtypeStruct((ROWS_TOTAL, 128), jnp.int32),
          jax.ShapeDtypeStruct((ROWS_TOTAL, 128), jnp.int32),
          jax.ShapeDtypeStruct((ROWS_TOTAL, 128), jnp.float32),
          jax.ShapeDtypeStruct((N, D), jnp.float32),
      ],
  )(edge_index, rt, af, wembp, bembp)


_BN = 1000


def _h1_body(nf_ref, odp_ref, w1_ref, h1_ref):
  od = odp_ref[:, 0] + odp_ref[:, 1]
  odi = lax.rsqrt(jnp.maximum(od, 1.0))
  h1_ref[...] = jnp.dot(nf_ref[...] * odi[:, None], w1_ref[...],
                        preferred_element_type=jnp.float32)


def _tc_h1(nf, odT, w1p):
  return pl.pallas_call(
      _h1_body,
      grid=(N // _BN,),
      in_specs=[
          pl.BlockSpec((_BN, D), lambda i: (i, 0)),
          pl.BlockSpec((_BN, NC), lambda i: (i, 0)),
          pl.BlockSpec((D, D), lambda i: (0, 0)),
      ],
      out_specs=pl.BlockSpec((_BN, D), lambda i: (i, 0)),
      out_shape=jax.ShapeDtypeStruct((N, D), jnp.float32),
  )(nf, odT, w1p)


def _h2_body(aggp_ref, odp_ref, idp_ref, b1_ref, w2r_ref, h2_ref, c_ref):
  agg = aggp_ref[0] + aggp_ref[1]
  od = odp_ref[:, 0] + odp_ref[:, 1]
  idg = idp_ref[:, 0] + idp_ref[:, 1]
  odi = lax.rsqrt(jnp.maximum(od, 1.0))
  idi = lax.rsqrt(jnp.maximum(idg, 1.0))
  x1 = jax.nn.relu(agg * idi[:, None] + b1_ref[...])
  h2 = jnp.sum((x1 * odi[:, None]) * w2r_ref[...], axis=1, keepdims=True)
  h2_ref[...] = h2
  c_ref[...] = idi[:, None]


def _tc_h2(aggp, odT, idT, b1p, w2row):
  return pl.pallas_call(
      _h2_body,
      grid=(N // _BN,),
      in_specs=[
          pl.BlockSpec((NC, _BN, D), lambda i: (0, i, 0)),
          pl.BlockSpec((_BN, NC), lambda i: (i, 0)),
          pl.BlockSpec((_BN, NC), lambda i: (i, 0)),
          pl.BlockSpec((1, D), lambda i: (0, 0)),
          pl.BlockSpec((1, D), lambda i: (0, 0)),
      ],
      out_specs=[
          pl.BlockSpec((_BN, 1), lambda i: (i, 0)),
          pl.BlockSpec((_BN, 1), lambda i: (i, 0)),
      ],
      out_shape=[
          jax.ShapeDtypeStruct((N_PAD, 1), jnp.float32),
          jax.ShapeDtypeStruct((N_PAD, 1), jnp.float32),
      ],
  )(aggp, odT, idT, b1p, w2row)


def kernel(atom_features, edge_index, r, W_emb, b_emb, W1, b1, W2, b2):
  wembp = jnp.pad(W_emb, ((0, 0), (0, D - 10)))
  bembp = jnp.pad(b_emb, (0, D - 10)).reshape(1, D)
  src2d, dst2d, ew2d, nf = _tc_prep(edge_index, jnp.swapaxes(r, 0, 1),
                                    atom_features, wembp, bembp)

  odp_full, idp_full = _make_dega()(src2d, dst2d)
  odT = odp_full[:, :N].T
  idT = idp_full[:, :N].T

  w1p = jnp.pad(W1, ((0, D - 10), (0, D - 10)))
  h1 = _tc_h1(nf, odT, w1p)

  (aggp,) = _make_scat()(src2d, dst2d, ew2d, h1)

  b1p = jnp.pad(b1, (0, D - 10)).reshape(1, D)
  w2row = jnp.pad(W2[:, 0], (0, D - 10)).reshape(1, D)
  h2, cd = _tc_h2(aggp, odT, idT, b1p, w2row)

  (parts,) = _make_esum()(src2d, dst2d, ew2d,
                          h2.reshape(N_PAD), cd.reshape(N_PAD))

  return jnp.sum(parts) / jnp.float32(N) + b2[0]


# speedup vs baseline: 1.0203x; 1.0203x over previous
import functools

import jax
import jax.numpy as jnp
from jax import lax
from jax.experimental import pallas as pl
from jax.experimental.pallas import tpu as pltpu
from jax.experimental.pallas import tpu_sc as plsc

N = 100000
E = 3200000
D = 16
NC = 2
NS = 16
NW = NC * NS
ROWS_TOTAL = E // 128
ROWS_PER_WORKER = 781
TAIL_ROW0 = ROWS_PER_WORKER * NW
TAIL_ROWS = ROWS_TOTAL - TAIL_ROW0
CHUNK_ROWS = 11
CHUNK_E = CHUNK_ROWS * 128
N_CHUNKS = ROWS_PER_WORKER // CHUNK_ROWS
N_PAD = 100096
TILE_NODES = N_PAD // NS

_mesh = functools.partial(
    plsc.VectorSubcoreMesh, core_axis_name="c", subcore_axis_name="s",
    num_cores=NC, num_subcores=NS)

_sc_params = functools.partial(
    pltpu.CompilerParams, use_tc_tiling_on_sc=False,
    needs_layout_passes=False)


def _worker_base():
  c = lax.axis_index("c")
  s = lax.axis_index("s")
  wid = c * NS + s
  return c, s, wid, wid * ROWS_PER_WORKER


def _dega_body(src_hbm, dst_hbm, od_hbm, id_hbm,
               srcb, dstb, onesb, zerob, od_sh, id_sh, hsem):
  c, s, wid, row0 = _worker_base()

  @pl.loop(0, TILE_NODES // 16)
  def _z(i):
    zerob[pl.ds(i * 16, 16)] = jnp.zeros((16,), jnp.float32)

  for g in range(8):
    onesb[pl.ds(g * 16, 16)] = jnp.ones((16,), jnp.float32)

  node0 = s * TILE_NODES
  pltpu.sync_copy(zerob, od_sh.at[pl.ds(node0, TILE_NODES)])
  pltpu.sync_copy(zerob, id_sh.at[pl.ds(node0, TILE_NODES)])
  plsc.subcore_barrier()

  def _do_rows(r0, nrows):
    pltpu.sync_copy(src_hbm.at[pl.ds(r0, nrows), :],
                    srcb.at[pl.ds(0, nrows), :])
    pltpu.sync_copy(dst_hbm.at[pl.ds(r0, nrows), :],
                    dstb.at[pl.ds(0, nrows), :])
    descs = []
    for j in range(nrows):
      descs.append(pltpu.async_copy(onesb, od_sh.at[srcb.at[j]], hsem,
                                    add=True))
      descs.append(pltpu.async_copy(onesb, id_sh.at[dstb.at[j]], hsem,
                                    add=True))
    for d in descs:
      d.wait()

  @pl.loop(0, N_CHUNKS)
  def _chunk(ch):
    _do_rows(row0 + ch * CHUNK_ROWS, CHUNK_ROWS)

  @pl.when(wid == 0)
  def _tail():
    _do_rows(TAIL_ROW0, TAIL_ROWS)

  plsc.subcore_barrier()
  pltpu.sync_copy(od_sh.at[pl.ds(node0, TILE_NODES)],
                  od_hbm.at[c, pl.ds(node0, TILE_NODES)])
  pltpu.sync_copy(id_sh.at[pl.ds(node0, TILE_NODES)],
                  id_hbm.at[c, pl.ds(node0, TILE_NODES)])


def _make_dega():
  return pl.kernel(
      _dega_body,
      out_type=[
          jax.ShapeDtypeStruct((NC, N_PAD), jnp.float32),
          jax.ShapeDtypeStruct((NC, N_PAD), jnp.float32),
      ],
      mesh=_mesh(),
      compiler_params=_sc_params(),
      scratch_types=[
          pltpu.VMEM((CHUNK_ROWS, 128), jnp.int32),
          pltpu.VMEM((CHUNK_ROWS, 128), jnp.int32),
          pltpu.VMEM((128,), jnp.float32),
          pltpu.VMEM((TILE_NODES,), jnp.float32),
          pltpu.VMEM_SHARED((N_PAD,), jnp.float32),
          pltpu.VMEM_SHARED((N_PAD,), jnp.float32),
          pltpu.SemaphoreType.DMA,
      ],
  )


def _scat_body(src_hbm, dst_hbm, ew_hbm, h1_hbm, agg_hbm,
               srcb, dstb, ewb, rows, agg_sh, gsem, ssem):
  c, s, wid, row0 = _worker_base()

  zstep = TILE_NODES // 16

  @pl.loop(0, zstep)
  def _z(i):
    rows[i, :] = jnp.zeros((16,), jnp.float32)

  node0 = s * TILE_NODES
  for k in range(16):
    pltpu.sync_copy(rows.at[pl.ds(0, zstep), :],
                    agg_sh.at[pl.ds(node0 + k * zstep, zstep), :])
  plsc.subcore_barrier()

  def _do_rows(r0, nrows):
    pltpu.sync_copy(src_hbm.at[pl.ds(r0, nrows), :],
                    srcb.at[pl.ds(0, nrows), :])
    pltpu.sync_copy(dst_hbm.at[pl.ds(r0, nrows), :],
                    dstb.at[pl.ds(0, nrows), :])
    pltpu.sync_copy(ew_hbm.at[pl.ds(r0, nrows), :],
                    ewb.at[pl.ds(0, nrows), :])
    gds = [pltpu.async_copy(h1_hbm.at[srcb.at[j]],
                            rows.at[pl.ds(j * 128, 128), :], gsem)
           for j in range(nrows)]
    sds = []
    for j in range(nrows):
      gds[j].wait()

      @pl.loop(0, 8)
      def _scale(q):
        base = j * 128 + q * 16
        ew16 = ewb[j, pl.ds(q * 16, 16)]
        for i in range(16):
          rows[base + i, :] = rows[base + i, :] * ew16[i]

      sds.append(pltpu.async_copy(rows.at[pl.ds(j * 128, 128), :],
                                  agg_sh.at[dstb.at[j]], ssem, add=True))
    for d in sds:
      d.wait()

  @pl.loop(0, N_CHUNKS)
  def _chunk(ch):
    _do_rows(row0 + ch * CHUNK_ROWS, CHUNK_ROWS)

  @pl.when(wid == 0)
  def _tail():
    _do_rows(TAIL_ROW0, TAIL_ROWS)

  plsc.subcore_barrier()
  for k in range(16):
    pltpu.sync_copy(agg_sh.at[pl.ds(node0 + k * zstep, zstep), :],
                    agg_hbm.at[c].at[pl.ds(node0 + k * zstep, zstep), :])


def _make_scat():
  return pl.kernel(
      _scat_body,
      out_type=[
          jax.ShapeDtypeStruct((NC, N_PAD, D), jnp.float32),
      ],
      mesh=_mesh(),
      compiler_params=_sc_params(),
      scratch_types=[
          pltpu.VMEM((CHUNK_ROWS, 128), jnp.int32),
          pltpu.VMEM((CHUNK_ROWS, 128), jnp.int32),
          pltpu.VMEM((CHUNK_ROWS, 128), jnp.float32),
          pltpu.VMEM((CHUNK_E, D), jnp.float32),
          pltpu.VMEM_SHARED((N_PAD, D), jnp.float32),
          pltpu.SemaphoreType.DMA,
          pltpu.SemaphoreType.DMA,
      ],
  )


def _esum_body(src_hbm, dst_hbm, ew_hbm, h2_hbm, cd_hbm, part_hbm,
               srcb, dstb, ewb, h2b, cb, accb, h2_sh, c_sh, gsem):
  c, s, wid, row0 = _worker_base()

  node0 = s * TILE_NODES
  pltpu.sync_copy(h2_hbm.at[pl.ds(node0, TILE_NODES)],
                  h2_sh.at[pl.ds(node0, TILE_NODES)])
  pltpu.sync_copy(cd_hbm.at[pl.ds(node0, TILE_NODES)],
                  c_sh.at[pl.ds(node0, TILE_NODES)])
  plsc.subcore_barrier()

  def _do_rows(r0, nrows, acc):
    pltpu.sync_copy(src_hbm.at[pl.ds(r0, nrows), :],
                    srcb.at[pl.ds(0, nrows), :])
    pltpu.sync_copy(dst_hbm.at[pl.ds(r0, nrows), :],
                    dstb.at[pl.ds(0, nrows), :])
    pltpu.sync_copy(ew_hbm.at[pl.ds(r0, nrows), :],
                    ewb.at[pl.ds(0, nrows), :])
    cds = []
    for j in range(nrows):
      cds.append((pltpu.async_copy(h2_sh.at[srcb.at[j]],
                                   h2b.at[pl.ds(j * 128, 128)], gsem),
                  pltpu.async_copy(c_sh.at[dstb.at[j]],
                                   cb.at[pl.ds(j * 128, 128)], gsem)))
    for j in range(nrows):
      cds[j][0].wait()
      cds[j][1].wait()
      for g in range(8):
        sl = pl.ds(j * 128 + g * 16, 16)
        acc = acc + ewb[j, pl.ds(g * 16, 16)] * h2b[sl] * cb[sl]
    return acc

  def _chunk(ch, acc):
    return _do_rows(row0 + ch * CHUNK_ROWS, CHUNK_ROWS, acc)

  acc = lax.fori_loop(0, N_CHUNKS, _chunk, jnp.zeros((16,), jnp.float32))
  acc2 = _do_rows(TAIL_ROW0, TAIL_ROWS, acc)
  accb[...] = jnp.where(wid == 0, acc2, acc)
  pltpu.sync_copy(accb, part_hbm.at[c, s, :])


def _make_esum():
  return pl.kernel(
      _esum_body,
      out_type=[
          jax.ShapeDtypeStruct((NC, NS, 16), jnp.float32),
      ],
      mesh=_mesh(),
      compiler_params=_sc_params(),
      scratch_types=[
          pltpu.VMEM((CHUNK_ROWS, 128), jnp.int32),
          pltpu.VMEM((CHUNK_ROWS, 128), jnp.int32),
          pltpu.VMEM((CHUNK_ROWS, 128), jnp.float32),
          pltpu.VMEM((CHUNK_E,), jnp.float32),
          pltpu.VMEM((CHUNK_E,), jnp.float32),
          pltpu.VMEM((16,), jnp.float32),
          pltpu.VMEM_SHARED((N_PAD,), jnp.float32),
          pltpu.VMEM_SHARED((N_PAD,), jnp.float32),
          pltpu.SemaphoreType.DMA,
      ],
  )


_BR_E = 5120
_ROWS_B = _BR_E // 128


def _prep_body(ei_ref, rt_ref, src_ref, dst_ref, ew_ref):
  src_ref[...] = ei_ref[0, :].reshape(_ROWS_B, 128)
  dst_ref[...] = ei_ref[1, :].reshape(_ROWS_B, 128)
  x = rt_ref[0, :]
  y = rt_ref[1, :]
  z = rt_ref[2, :]
  ew = jnp.exp(-(x * x + y * y + z * z))
  ew_ref[...] = ew.reshape(_ROWS_B, 128)


def _tc_prep(edge_index, rt):
  return pl.pallas_call(
      _prep_body,
      grid=(E // _BR_E,),
      in_specs=[
          pl.BlockSpec((2, _BR_E), lambda i: (0, i)),
          pl.BlockSpec((3, _BR_E), lambda i: (0, i)),
      ],
      out_specs=[
          pl.BlockSpec((_ROWS_B, 128), lambda i: (i, 0)),
          pl.BlockSpec((_ROWS_B, 128), lambda i: (i, 0)),
          pl.BlockSpec((_ROWS_B, 128), lambda i: (i, 0)),
      ],
      out_shape=[
          jax.ShapeDtypeStruct((ROWS_TOTAL, 128), jnp.int32),
          jax.ShapeDtypeStruct((ROWS_TOTAL, 128), jnp.int32),
          jax.ShapeDtypeStruct((ROWS_TOTAL, 128), jnp.float32),
      ],
  )(edge_index, rt)


_BN = 1000


def _nf_body(af_ref, wemb_ref, bemb_ref, nf_ref):
  nf_ref[...] = jnp.dot(af_ref[...], wemb_ref[...],
                        preferred_element_type=jnp.float32) + bemb_ref[...]


def _tc_nf(af, wembp, bembp):
  return pl.pallas_call(
      _nf_body,
      grid=(N // _BN,),
      in_specs=[
          pl.BlockSpec((_BN, 128), lambda i: (i, 0)),
          pl.BlockSpec((128, D), lambda i: (0, 0)),
          pl.BlockSpec((1, D), lambda i: (0, 0)),
      ],
      out_specs=pl.BlockSpec((_BN, D), lambda i: (i, 0)),
      out_shape=jax.ShapeDtypeStruct((N, D), jnp.float32),
  )(af, wembp, bembp)


def _h1_body(nf_ref, odp_ref, w1_ref, h1_ref):
  od = odp_ref[:, 0] + odp_ref[:, 1]
  odi = lax.rsqrt(jnp.maximum(od, 1.0))
  h1_ref[...] = jnp.dot(nf_ref[...] * odi[:, None], w1_ref[...],
                        preferred_element_type=jnp.float32)


def _tc_h1(nf, odT, w1p):
  return pl.pallas_call(
      _h1_body,
      grid=(N // _BN,),
      in_specs=[
          pl.BlockSpec((_BN, D), lambda i: (i, 0)),
          pl.BlockSpec((_BN, NC), lambda i: (i, 0)),
          pl.BlockSpec((D, D), lambda i: (0, 0)),
      ],
      out_specs=pl.BlockSpec((_BN, D), lambda i: (i, 0)),
      out_shape=jax.ShapeDtypeStruct((N, D), jnp.float32),
  )(nf, odT, w1p)


def _h2_body(aggp_ref, odp_ref, idp_ref, b1_ref, w2r_ref, h2_ref, c_ref):
  agg = aggp_ref[0] + aggp_ref[1]
  od = odp_ref[:, 0] + odp_ref[:, 1]
  idg = idp_ref[:, 0] + idp_ref[:, 1]
  odi = lax.rsqrt(jnp.maximum(od, 1.0))
  idi = lax.rsqrt(jnp.maximum(idg, 1.0))
  x1 = jax.nn.relu(agg * idi[:, None] + b1_ref[...])
  h2 = jnp.sum((x1 * odi[:, None]) * w2r_ref[...], axis=1, keepdims=True)
  h2_ref[...] = h2
  c_ref[...] = idi[:, None]


def _tc_h2(aggp, odT, idT, b1p, w2row):
  return pl.pallas_call(
      _h2_body,
      grid=(N // _BN,),
      in_specs=[
          pl.BlockSpec((NC, _BN, D), lambda i: (0, i, 0)),
          pl.BlockSpec((_BN, NC), lambda i: (i, 0)),
          pl.BlockSpec((_BN, NC), lambda i: (i, 0)),
          pl.BlockSpec((1, D), lambda i: (0, 0)),
          pl.BlockSpec((1, D), lambda i: (0, 0)),
      ],
      out_specs=[
          pl.BlockSpec((_BN, 1), lambda i: (i, 0)),
          pl.BlockSpec((_BN, 1), lambda i: (i, 0)),
      ],
      out_shape=[
          jax.ShapeDtypeStruct((N_PAD, 1), jnp.float32),
          jax.ShapeDtypeStruct((N_PAD, 1), jnp.float32),
      ],
  )(aggp, odT, idT, b1p, w2row)


def kernel(atom_features, edge_index, r, W_emb, b_emb, W1, b1, W2, b2):
  src2d, dst2d, ew2d = _tc_prep(edge_index, jnp.swapaxes(r, 0, 1))

  wembp = jnp.pad(W_emb, ((0, 0), (0, D - 10)))
  bembp = jnp.pad(b_emb, (0, D - 10)).reshape(1, D)
  nf = _tc_nf(atom_features, wembp, bembp)

  odp_full, idp_full = _make_dega()(src2d, dst2d)
  odT = odp_full[:, :N].T
  idT = idp_full[:, :N].T

  w1p = jnp.pad(W1, ((0, D - 10), (0, D - 10)))
  h1 = _tc_h1(nf, odT, w1p)

  (aggp,) = _make_scat()(src2d, dst2d, ew2d, h1)

  b1p = jnp.pad(b1, (0, D - 10)).reshape(1, D)
  w2row = jnp.pad(W2[:, 0], (0, D - 10)).reshape(1, D)
  h2, cd = _tc_h2(aggp, odT, idT, b1p, w2row)

  (parts,) = _make_esum()(src2d, dst2d, ew2d,
                          h2.reshape(N_PAD), cd.reshape(N_PAD))

  return jnp.sum(parts) / jnp.float32(N) + b2[0]
